# baseline (speedup 1.0000x reference)
import jax
import jax.numpy as jnp
from jax.experimental import pallas as pl

_ROWS_PER_BLOCK = 8
_N = 32768


def _masked_softmax_block(x_ref, m_ref, o_ref):
    x = x_ref[...]
    m = m_ref[...] == 1
    neg = jnp.finfo(x.dtype).min
    z = jnp.where(m, x, neg)
    mx = jnp.max(z, axis=-1, keepdims=True)
    e = jnp.where(m, jnp.exp(z - mx), jnp.zeros((), dtype=x.dtype))
    s = jnp.sum(e, axis=-1, keepdims=True)
    o_ref[...] = e / jnp.maximum(s, jnp.asarray(1e-30, dtype=x.dtype))


def kernel(inputLayer, mask):
    rows, cols = inputLayer.shape
    grid = (rows // _ROWS_PER_BLOCK,)
    spec = pl.BlockSpec((_ROWS_PER_BLOCK, cols), lambda i: (i, 0))
    return pl.pallas_call(
        _masked_softmax_block,
        grid=grid,
        in_specs=[spec, spec],
        out_specs=spec,
        out_shape=jax.ShapeDtypeStruct((rows, cols), inputLayer.dtype),
    )(inputLayer, mask)


# speedup vs baseline: 1.3533x; 1.3533x over previous
import jax
import jax.numpy as jnp
from jax.experimental import pallas as pl

_ROWS_PER_BLOCK = 32
_N = 32768


def _masked_softmax_block(x_ref, m_ref, o_ref):
    x = x_ref[...]
    m = m_ref[...] == 1
    neg = jnp.finfo(x.dtype).min
    z = jnp.where(m, x, neg)
    mx = jnp.max(z, axis=-1, keepdims=True)
    e = jnp.where(m, jnp.exp(z - mx), jnp.zeros((), dtype=x.dtype))
    s = jnp.sum(e, axis=-1, keepdims=True)
    o_ref[...] = e / jnp.maximum(s, jnp.asarray(1e-30, dtype=x.dtype))


def kernel(inputLayer, mask):
    rows, cols = inputLayer.shape
    grid = (rows // _ROWS_PER_BLOCK,)
    spec = pl.BlockSpec((_ROWS_PER_BLOCK, cols), lambda i: (i, 0))
    return pl.pallas_call(
        _masked_softmax_block,
        grid=grid,
        in_specs=[spec, spec],
        out_specs=spec,
        out_shape=jax.ShapeDtypeStruct((rows, cols), inputLayer.dtype),
    )(inputLayer, mask)


# speedup vs baseline: 1.3786x; 1.0187x over previous
import jax
import jax.numpy as jnp
from jax.experimental import pallas as pl

_ROWS_PER_BLOCK = 32
_N = 32768


def _masked_softmax_block(x_ref, m_ref, o_ref):
    x = x_ref[...]
    m = m_ref[...] == 1
    neg = jnp.finfo(x.dtype).min
    z = jnp.where(m, x, neg)
    mx = jnp.max(z, axis=-1, keepdims=True)
    e = jnp.exp(z - mx)
    s = jnp.sum(e, axis=-1, keepdims=True)
    scale = jnp.where(
        mx > neg,
        jnp.asarray(1.0, x.dtype) / jnp.maximum(s, jnp.asarray(1e-30, x.dtype)),
        jnp.zeros((), x.dtype),
    )
    o_ref[...] = e * scale


def kernel(inputLayer, mask):
    rows, cols = inputLayer.shape
    grid = (rows // _ROWS_PER_BLOCK,)
    spec = pl.BlockSpec((_ROWS_PER_BLOCK, cols), lambda i: (i, 0))
    return pl.pallas_call(
        _masked_softmax_block,
        grid=grid,
        in_specs=[spec, spec],
        out_specs=spec,
        out_shape=jax.ShapeDtypeStruct((rows, cols), inputLayer.dtype),
    )(inputLayer, mask)


# speedup vs baseline: 1.3949x; 1.0118x over previous
import jax
import jax.numpy as jnp
from jax.experimental import pallas as pl

_ROWS_PER_BLOCK = 64
_N = 32768


def _masked_softmax_block(x_ref, m_ref, o_ref):
    x = x_ref[...]
    m = m_ref[...] == 1
    neg = jnp.finfo(x.dtype).min
    z = jnp.where(m, x, neg)
    mx = jnp.max(z, axis=-1, keepdims=True)
    e = jnp.exp(z - mx)
    s = jnp.sum(e, axis=-1, keepdims=True)
    scale = jnp.where(
        mx > neg,
        jnp.asarray(1.0, x.dtype) / jnp.maximum(s, jnp.asarray(1e-30, x.dtype)),
        jnp.zeros((), x.dtype),
    )
    o_ref[...] = e * scale


def kernel(inputLayer, mask):
    rows, cols = inputLayer.shape
    grid = (rows // _ROWS_PER_BLOCK,)
    spec = pl.BlockSpec((_ROWS_PER_BLOCK, cols), lambda i: (i, 0))
    return pl.pallas_call(
        _masked_softmax_block,
        grid=grid,
        in_specs=[spec, spec],
        out_specs=spec,
        out_shape=jax.ShapeDtypeStruct((rows, cols), inputLayer.dtype),
    )(inputLayer, mask)
